# baseline (speedup 1.0000x reference)
import jax
import jax.numpy as jnp
from jax import lax
from jax.experimental import pallas as pl
from jax.experimental.pallas import tpu as pltpu

_NUM_CLASSES = 21
_THRESHOLD = 0.5
_NEG_POS = 3
_VAR0 = 0.1
_VAR1 = 0.2

_B = 32
_P = 8732
_ROWS = 8
_LANES = 1152
_PP = _ROWS * _LANES
_O = 8
_MAX_FINITE_BITS = 0x7F7FFFFF


def _mbl_kernel(loc_ref, conf_ref, priors_ref, targets_ref, out_ref,
                bits_ref, meta_ref):
    b = pl.program_id(0)

    @pl.when(b == 0)
    def _init():
        out_ref[...] = jnp.zeros_like(out_ref)

    shape = (_ROWS, _LANES)
    row_i = lax.broadcasted_iota(jnp.int32, shape, 0)
    col_i = lax.broadcasted_iota(jnp.int32, shape, 1)
    gidx = row_i * _LANES + col_i
    valid = gidx < _P

    pr = priors_ref[...]
    px, py, pw, ph = pr[0], pr[1], pr[2], pr[3]
    pf_x0 = px - pw / 2.0
    pf_y0 = py - ph / 2.0
    pf_x1 = px + pw / 2.0
    pf_y1 = py + ph / 2.0
    area_p = (pf_x1 - pf_x0) * (pf_y1 - pf_y0)

    t = targets_ref[0]

    neg1 = jnp.float32(-1.0)
    cmax = jnp.full(shape, neg1)
    ious = []
    bpi = []
    for o in range(_O):
        tx0 = t[o, 0]
        ty0 = t[o, 1]
        tx1 = t[o, 2]
        ty1 = t[o, 3]
        ix0 = jnp.maximum(pf_x0, tx0)
        iy0 = jnp.maximum(pf_y0, ty0)
        ix1 = jnp.minimum(pf_x1, tx1)
        iy1 = jnp.minimum(pf_y1, ty1)
        iw = jnp.maximum(ix1 - ix0, 0.0)
        ih = jnp.maximum(iy1 - iy0, 0.0)
        inter = iw * ih
        area_t = (tx1 - tx0) * (ty1 - ty0)
        iou = inter / (area_t + area_p - inter)
        ious.append(iou)
        cmax = jnp.maximum(cmax, iou)
        m = jnp.max(iou)
        bpi.append(jnp.min(jnp.where(iou == m, gidx, _PP)))

    bti = jnp.full(shape, _O, dtype=jnp.int32)
    for o in range(_O - 1, -1, -1):
        bti = jnp.where(ious[o] == cmax, o, bti)
    bto = cmax

    for o in range(_O):
        hit = gidx == bpi[o]
        bto = jnp.where(hit, 2.0, bto)
        bti = jnp.where(hit, o, bti)

    lab = jnp.zeros(shape, dtype=jnp.int32)
    mx0 = jnp.zeros(shape, dtype=jnp.float32)
    my0 = jnp.zeros(shape, dtype=jnp.float32)
    mx1 = jnp.zeros(shape, dtype=jnp.float32)
    my1 = jnp.zeros(shape, dtype=jnp.float32)
    for o in range(_O):
        sel = bti == o
        lab = jnp.where(sel, t[o, 4].astype(jnp.int32), lab)
        mx0 = jnp.where(sel, t[o, 0], mx0)
        my0 = jnp.where(sel, t[o, 1], my0)
        mx1 = jnp.where(sel, t[o, 2], mx1)
        my1 = jnp.where(sel, t[o, 3], my1)
    conf_t = jnp.where(bto < _THRESHOLD, 0, lab + 1)
    pos = conf_t > 0

    g_cx = ((mx0 + mx1) / 2.0 - px) / (_VAR0 * pw)
    g_cy = ((my0 + my1) / 2.0 - py) / (_VAR0 * ph)
    g_w = jnp.log((mx1 - mx0) / pw) / _VAR1
    g_h = jnp.log((my1 - my0) / ph) / _VAR1

    lc = loc_ref[0]
    sl1 = jnp.zeros(shape, dtype=jnp.float32)
    for c, g in enumerate((g_cx, g_cy, g_w, g_h)):
        d = lc[c] - g
        ad = jnp.abs(d)
        sl1 = sl1 + jnp.where(ad < 1.0, 0.5 * d * d, ad - 0.5)
    loss_l_b = jnp.sum(jnp.where(pos, sl1, 0.0))

    cf = conf_ref[0]
    mxl = cf[0]
    for c in range(1, _NUM_CLASSES):
        mxl = jnp.maximum(mxl, cf[c])
    ssum = jnp.zeros(shape, dtype=jnp.float32)
    gathered = jnp.zeros(shape, dtype=jnp.float32)
    for c in range(_NUM_CLASSES):
        ssum = ssum + jnp.exp(cf[c] - mxl)
        gathered = jnp.where(conf_t == c, cf[c], gathered)
    ce = jnp.log(ssum) + mxl - gathered

    num_pos = jnp.sum(jnp.where(pos, 1, 0))

    mine = jnp.where(pos | jnp.logical_not(valid), 0.0, ce)
    bits_ref[pl.ds(b, 1)] = lax.bitcast_convert_type(mine, jnp.int32)[None]
    meta_ref[pl.ds(b, 1)] = jnp.full((1, 1, 128), num_pos.astype(jnp.float32))

    pos_ce_b = jnp.sum(jnp.where(pos, ce, 0.0))

    acc_i = lax.broadcasted_iota(jnp.int32, (1, 1, 8), 2)
    vec = (jnp.where(acc_i == 0, loss_l_b, 0.0)
           + jnp.where(acc_i == 1, pos_ce_b, 0.0)
           + jnp.where(acc_i == 2, num_pos.astype(jnp.float32), 0.0))
    out_ref[...] = out_ref[...] + vec

    @pl.when(b == _B - 1)
    def _mining():
        bits = bits_ref[...]
        npos = meta_ref[...][:, :, 0:1].astype(jnp.int32)
        k = jnp.minimum(_NEG_POS * npos, _P - 1)

        def body(_, lohi):
            lo, hi = lohi
            mid = lo + (hi - lo + 1) // 2
            cnt = jnp.sum(jnp.where(bits >= mid, 1, 0), axis=(1, 2),
                          keepdims=True)
            ge = cnt >= k
            return jnp.where(ge, mid, lo), jnp.where(ge, hi, mid - 1)

        lo0 = jnp.zeros((_B, 1, 1), jnp.int32)
        hi0 = jnp.full((_B, 1, 1), _MAX_FINITE_BITS, jnp.int32)
        lo, _ = lax.fori_loop(0, 31, body, (lo0, hi0))
        vals = lax.bitcast_convert_type(bits, jnp.float32)
        gt = bits > lo
        cnt_gt = jnp.sum(jnp.where(gt, 1, 0), axis=(1, 2), keepdims=True)
        sum_gt = jnp.sum(jnp.where(gt, vals, 0.0), axis=(1, 2), keepdims=True)
        t_val = jnp.max(jnp.where(bits == lo, vals, 0.0), axis=(1, 2),
                        keepdims=True)
        topk = sum_gt + (k - cnt_gt).astype(jnp.float32) * t_val
        total = jnp.sum(topk)

        acc_j = lax.broadcasted_iota(jnp.int32, (1, 1, 8), 2)
        out_ref[...] = out_ref[...] + jnp.where(acc_j == 1, total, 0.0)


@jax.jit
def kernel(loc_data, conf_data, priors, targets):
    B = loc_data.shape[0]
    pad = _PP - _P

    loc_p = jnp.pad(jnp.transpose(loc_data, (0, 2, 1)),
                    ((0, 0), (0, 0), (0, pad))).reshape(B, 4, _ROWS, _LANES)
    conf_p = jnp.pad(jnp.transpose(conf_data, (0, 2, 1)),
                     ((0, 0), (0, 0), (0, pad))).reshape(B, _NUM_CLASSES, _ROWS, _LANES)
    pad_cols = jnp.tile(jnp.array([[-10.0], [-10.0], [1.0], [1.0]], jnp.float32),
                        (1, pad))
    priors_p = jnp.concatenate([priors.T, pad_cols], axis=1).reshape(4, _ROWS, _LANES)

    out = pl.pallas_call(
        _mbl_kernel,
        grid=(B,),
        in_specs=[
            pl.BlockSpec((1, 4, _ROWS, _LANES), lambda b: (b, 0, 0, 0)),
            pl.BlockSpec((1, _NUM_CLASSES, _ROWS, _LANES), lambda b: (b, 0, 0, 0)),
            pl.BlockSpec((4, _ROWS, _LANES), lambda b: (0, 0, 0)),
            pl.BlockSpec((1, _O, 5), lambda b: (b, 0, 0)),
        ],
        out_specs=pl.BlockSpec((1, 1, 8), lambda b: (0, 0, 0)),
        out_shape=jax.ShapeDtypeStruct((1, 1, 8), jnp.float32),
        scratch_shapes=[
            pltpu.VMEM((_B, _ROWS, _LANES), jnp.int32),
            pltpu.VMEM((_B, 1, 128), jnp.float32),
        ],
    )(loc_p, conf_p, priors_p, targets)

    s = out[0, 0]
    n = jnp.maximum(s[2], 1.0)
    return s[0] / n, s[1] / n


# speedup vs baseline: 1.0408x; 1.0408x over previous
import jax
import jax.numpy as jnp
from jax import lax
from jax.experimental import pallas as pl
from jax.experimental.pallas import tpu as pltpu

_NUM_CLASSES = 21
_THRESHOLD = 0.5
_NEG_POS = 3
_VAR0 = 0.1
_VAR1 = 0.2

_B = 32
_G = 4
_P = 8732
_ROWS = 8
_LANES = 1152
_PP = _ROWS * _LANES
_O = 8
_MAX_FINITE_BITS = 0x7F7FFFFF
_SIGN_BIT = -2147483648
_ABS_MASK = 0x7FFFFFFF


def _one_image(loc, conf, tgt, pf, gidx, valid):
    pf_x0, pf_y0, pf_x1, pf_y1, area_p, px, py, pw, ph = pf
    shape = (_ROWS, _LANES)
    t = tgt

    cmax = jnp.full(shape, jnp.float32(-1.0))
    ious = []
    bpi = []
    for o in range(_O):
        tx0 = t[o, 0]
        ty0 = t[o, 1]
        tx1 = t[o, 2]
        ty1 = t[o, 3]
        ix0 = jnp.maximum(pf_x0, tx0)
        iy0 = jnp.maximum(pf_y0, ty0)
        ix1 = jnp.minimum(pf_x1, tx1)
        iy1 = jnp.minimum(pf_y1, ty1)
        iw = jnp.maximum(ix1 - ix0, 0.0)
        ih = jnp.maximum(iy1 - iy0, 0.0)
        inter = iw * ih
        area_t = (tx1 - tx0) * (ty1 - ty0)
        iou = inter / (area_t + area_p - inter)
        ious.append(iou)
        cmax = jnp.maximum(cmax, iou)
        m = jnp.max(iou)
        bpi.append(jnp.min(jnp.where(iou == m, gidx, _PP)))

    bti = jnp.full(shape, _O, dtype=jnp.int32)
    for o in range(_O - 1, -1, -1):
        bti = jnp.where(ious[o] == cmax, o, bti)
    bto = cmax

    for o in range(_O):
        hit = gidx == bpi[o]
        bto = jnp.where(hit, 2.0, bto)
        bti = jnp.where(hit, o, bti)

    lab = jnp.zeros(shape, dtype=jnp.int32)
    mx0 = jnp.zeros(shape, dtype=jnp.float32)
    my0 = jnp.zeros(shape, dtype=jnp.float32)
    mx1 = jnp.zeros(shape, dtype=jnp.float32)
    my1 = jnp.zeros(shape, dtype=jnp.float32)
    for o in range(_O):
        sel = bti == o
        lab = jnp.where(sel, t[o, 4].astype(jnp.int32), lab)
        mx0 = jnp.where(sel, t[o, 0], mx0)
        my0 = jnp.where(sel, t[o, 1], my0)
        mx1 = jnp.where(sel, t[o, 2], mx1)
        my1 = jnp.where(sel, t[o, 3], my1)
    conf_t = jnp.where(bto < _THRESHOLD, 0, lab + 1)
    pos = conf_t > 0

    g_cx = ((mx0 + mx1) / 2.0 - px) / (_VAR0 * pw)
    g_cy = ((my0 + my1) / 2.0 - py) / (_VAR0 * ph)
    g_w = jnp.log((mx1 - mx0) / pw) / _VAR1
    g_h = jnp.log((my1 - my0) / ph) / _VAR1
    sl1 = jnp.zeros(shape, dtype=jnp.float32)
    for c, g in enumerate((g_cx, g_cy, g_w, g_h)):
        d = loc[c] - g
        ad = jnp.abs(d)
        sl1 = sl1 + jnp.where(ad < 1.0, 0.5 * d * d, ad - 0.5)
    sl1 = jnp.where(pos, sl1, 0.0)

    mxl = conf[0]
    for c in range(1, _NUM_CLASSES):
        mxl = jnp.maximum(mxl, conf[c])
    ssum = jnp.zeros(shape, dtype=jnp.float32)
    gathered = jnp.zeros(shape, dtype=jnp.float32)
    for c in range(_NUM_CLASSES):
        ssum = ssum + jnp.exp(conf[c] - mxl)
        gathered = jnp.where(conf_t == c, conf[c], gathered)
    ce = jnp.log(ssum) + mxl - gathered

    ce_bits = lax.bitcast_convert_type(jnp.where(valid, ce, 0.0), jnp.int32)
    stored = jnp.where(pos & valid, ce_bits | jnp.int32(_SIGN_BIT), ce_bits)
    return sl1, stored


def _mbl_kernel(loc_ref, conf_ref, priors_ref, targets_ref, out_ref,
                bits_ref, acc_ref):
    i = pl.program_id(0)

    @pl.when(i == 0)
    def _init():
        acc_ref[...] = jnp.zeros_like(acc_ref)

    shape = (_ROWS, _LANES)
    row_i = lax.broadcasted_iota(jnp.int32, shape, 0)
    col_i = lax.broadcasted_iota(jnp.int32, shape, 1)
    gidx = row_i * _LANES + col_i
    valid = gidx < _P

    pr = priors_ref[...]
    px, py, pw, ph = pr[0], pr[1], pr[2], pr[3]
    pf_x0 = px - pw / 2.0
    pf_y0 = py - ph / 2.0
    pf_x1 = px + pw / 2.0
    pf_y1 = py + ph / 2.0
    area_p = (pf_x1 - pf_x0) * (pf_y1 - pf_y0)
    pf = (pf_x0, pf_y0, pf_x1, pf_y1, area_p, px, py, pw, ph)

    acc = acc_ref[...]
    for g in range(_G):
        sl1, stored = _one_image(loc_ref[g], conf_ref[g], targets_ref[g],
                                 pf, gidx, valid)
        acc = acc + sl1
        bits_ref[pl.ds(i * _G + g, 1)] = stored[None]
    acc_ref[...] = acc

    @pl.when(i == _B // _G - 1)
    def _mining():
        stored = bits_ref[...]
        posm = stored < 0
        cbits = stored & jnp.int32(_ABS_MASK)
        vals = lax.bitcast_convert_type(cbits, jnp.float32)
        mine = jnp.where(posm, 0, cbits)

        npos = jnp.sum(jnp.where(posm, 1, 0), axis=(1, 2), keepdims=True)
        k = jnp.minimum(_NEG_POS * npos, _P - 1)

        def body(_, lohi):
            lo, hi = lohi
            mid = lo + (hi - lo + 1) // 2
            cnt = jnp.sum(jnp.where(mine >= mid, 1, 0), axis=(1, 2),
                          keepdims=True)
            ge = cnt >= k
            return jnp.where(ge, mid, lo), jnp.where(ge, hi, mid - 1)

        lo0 = jnp.zeros((_B, 1, 1), jnp.int32)
        hi0 = jnp.full((_B, 1, 1), _MAX_FINITE_BITS, jnp.int32)
        lo, _ = lax.fori_loop(0, 31, body, (lo0, hi0))
        gt = mine > lo
        cnt_gt = jnp.sum(jnp.where(gt, 1, 0), axis=(1, 2), keepdims=True)
        sum_gt = jnp.sum(jnp.where(gt, vals, 0.0), axis=(1, 2), keepdims=True)
        t_val = jnp.max(
            jnp.where(mine == lo, lax.bitcast_convert_type(mine, jnp.float32),
                      0.0),
            axis=(1, 2), keepdims=True)
        topk = sum_gt + (k - cnt_gt).astype(jnp.float32) * t_val

        loss_c = jnp.sum(topk) + jnp.sum(jnp.where(posm, vals, 0.0))
        loss_l = jnp.sum(acc_ref[...])
        n_pos = jnp.sum(npos).astype(jnp.float32)

        acc_i = lax.broadcasted_iota(jnp.int32, (1, 1, 8), 2)
        out_ref[...] = (jnp.where(acc_i == 0, loss_l, 0.0)
                        + jnp.where(acc_i == 1, loss_c, 0.0)
                        + jnp.where(acc_i == 2, n_pos, 0.0))


@jax.jit
def kernel(loc_data, conf_data, priors, targets):
    B = loc_data.shape[0]
    pad = _PP - _P

    loc_p = jnp.pad(jnp.transpose(loc_data, (0, 2, 1)),
                    ((0, 0), (0, 0), (0, pad))).reshape(B, 4, _ROWS, _LANES)
    conf_p = jnp.pad(jnp.transpose(conf_data, (0, 2, 1)),
                     ((0, 0), (0, 0), (0, pad))).reshape(B, _NUM_CLASSES, _ROWS, _LANES)
    pad_cols = jnp.tile(jnp.array([[-10.0], [-10.0], [1.0], [1.0]], jnp.float32),
                        (1, pad))
    priors_p = jnp.concatenate([priors.T, pad_cols], axis=1).reshape(4, _ROWS, _LANES)

    out = pl.pallas_call(
        _mbl_kernel,
        grid=(B // _G,),
        in_specs=[
            pl.BlockSpec((_G, 4, _ROWS, _LANES), lambda i: (i, 0, 0, 0)),
            pl.BlockSpec((_G, _NUM_CLASSES, _ROWS, _LANES), lambda i: (i, 0, 0, 0)),
            pl.BlockSpec((4, _ROWS, _LANES), lambda i: (0, 0, 0)),
            pl.BlockSpec((_G, _O, 5), lambda i: (i, 0, 0)),
        ],
        out_specs=pl.BlockSpec((1, 1, 8), lambda i: (0, 0, 0)),
        out_shape=jax.ShapeDtypeStruct((1, 1, 8), jnp.float32),
        scratch_shapes=[
            pltpu.VMEM((_B, _ROWS, _LANES), jnp.int32),
            pltpu.VMEM((_ROWS, _LANES), jnp.float32),
        ],
    )(loc_p, conf_p, priors_p, targets)

    s = out[0, 0]
    n = jnp.maximum(s[2], 1.0)
    return s[0] / n, s[1] / n
